# baseline (speedup 1.0000x reference)
import functools

import numpy as np
import jax
import jax.numpy as jnp
from jax import lax
from jax.experimental import pallas as pl
from jax.experimental.pallas import tpu as pltpu
from jax.experimental.pallas import tpu_sc as plsc

B, S, V, D = 1024, 50, 100000, 128
N = B * S

_NC, _NS = 2, 16
NW = _NC * _NS
CH = 80

K_SLICES = 2
NSL = N // K_SLICES
S_SL = S // K_SLICES


def _make_pe():
    position = np.arange(S)[:, None].astype(np.float64)
    div_term = np.exp(np.arange(0, D, 2).astype(np.float64) * (-np.log(10000.0) / D))
    pe = np.zeros((S, D), dtype=np.float32)
    pe[:, 0::2] = np.sin(position * div_term)
    pe[:, 1::2] = np.cos(position * div_term)
    return pe


def _sc_gather3(tok_tab, act_tab, res_tab, tok_idx, act_idx, res_idx, n_rows):
    per_w = n_rows // NW
    nch = per_w // CH
    assert per_w % CH == 0 and nch % 2 == 0
    mesh = plsc.VectorSubcoreMesh(
        core_axis_name="c", subcore_axis_name="s",
        num_cores=_NC, num_subcores=_NS)

    @functools.partial(
        pl.kernel,
        out_type=(jax.ShapeDtypeStruct((n_rows, D), jnp.float32),) * 3,
        mesh=mesh,
        scratch_types=[
            pltpu.VMEM((per_w,), jnp.int32),
            pltpu.VMEM((CH, D), jnp.float32),
            pltpu.VMEM((CH, D), jnp.float32),
            pltpu.SemaphoreType.DMA,
            pltpu.SemaphoreType.DMA,
            pltpu.SemaphoreType.DMA,
        ],
    )
    def gather_k(tok_tab, act_tab, res_tab, tok_i, act_i, res_i,
                 o_tok, o_act, o_res, idx_v, buf_a, buf_b, sem_g, sem_sa, sem_sb):
        wid = lax.axis_index("s") * _NC + lax.axis_index("c")
        base = wid * per_w
        for tab, idx_hbm, out_hbm in ((tok_tab, tok_i, o_tok),
                                      (act_tab, act_i, o_act),
                                      (res_tab, res_i, o_res)):
            pltpu.sync_copy(idx_hbm.at[pl.ds(base, per_w)], idx_v)

            def gather(c, buf, tab=tab):
                pltpu.async_copy(
                    tab.at[idx_v.at[pl.ds(c * CH, CH)]], buf, sem_g).wait()

            def store_start(c, buf, sem, out_hbm=out_hbm):
                pltpu.async_copy(buf, out_hbm.at[pl.ds(base + c * CH, CH)], sem)

            def store_wait(c, buf, sem, out_hbm=out_hbm):
                pltpu.make_async_copy(
                    buf, out_hbm.at[pl.ds(base + c * CH, CH)], sem).wait()

            gather(0, buf_a)
            store_start(0, buf_a, sem_sa)
            gather(1, buf_b)
            store_start(1, buf_b, sem_sb)

            def pair(p, carry, tab=tab, out_hbm=out_hbm):
                c0 = 2 * p
                store_wait(c0 - 2, buf_a, sem_sa)
                gather(c0, buf_a)
                store_start(c0, buf_a, sem_sa)
                store_wait(c0 - 1, buf_b, sem_sb)
                gather(c0 + 1, buf_b)
                store_start(c0 + 1, buf_b, sem_sb)
                return carry

            lax.fori_loop(1, nch // 2, pair, 0)
            store_wait(nch - 2, buf_a, sem_sa)
            store_wait(nch - 1, buf_b, sem_sb)

    return gather_k(tok_tab, act_tab, res_tab, tok_idx, act_idx, res_idx)


def _fold_body(nW2, tW2, pW, nb2, tb2, cn_ref, ct_ref, bvec_ref):
    wn = pW[2 * D:3 * D, :]
    wt = pW[3 * D:4 * D, :]
    cn_ref[...] = jnp.dot(nW2[...], wn, preferred_element_type=jnp.float32)
    ct_ref[...] = jnp.dot(tW2[...], wt, preferred_element_type=jnp.float32)
    bvec_ref[...] = (jnp.dot(nb2[...], wn, preferred_element_type=jnp.float32)
                     + jnp.dot(tb2[...], wt, preferred_element_type=jnp.float32))


def _fold(num_W2, time_W2, proj_W, num_b2, time_b2):
    return pl.pallas_call(
        _fold_body,
        out_shape=(jax.ShapeDtypeStruct((D // 2, D), jnp.float32),
                   jax.ShapeDtypeStruct((D // 2, D), jnp.float32),
                   jax.ShapeDtypeStruct((1, D), jnp.float32)),
    )(num_W2, time_W2, proj_W, num_b2, time_b2)


NR = 1024


def _tc_body(aux_ref, tok_ref, act_ref, res_ref, nW1, nb1, tW1, tb1,
             pW2, cn, ct, bvec, pe_ref, out_ref):
    a = aux_ref[...]
    m = a[:, 0:1]
    nf = a[:, 1:2]
    t0 = a[:, 2:3]
    t1 = a[:, 3:4]
    h_n = jnp.maximum(nf * nW1[0:1, :] + nb1[0:1, :], 0.0)
    h_t = jnp.maximum(t0 * tW1[0:1, :] + t1 * tW1[1:2, :] + tb1[0:1, :], 0.0)
    w = pW2[...]
    p = (jnp.dot(act_ref[...], w[0:D, :], preferred_element_type=jnp.float32)
         + jnp.dot(res_ref[...], w[D:2 * D, :], preferred_element_type=jnp.float32)
         + jnp.dot(h_n, cn[...], preferred_element_type=jnp.float32)
         + jnp.dot(h_t, ct[...], preferred_element_type=jnp.float32)
         + bvec[0:1, :])
    out_ref[...] = m * p + tok_ref[...] + pe_ref[0]


def _tc_fuse(aux, tok_rows, act_rows, res_rows,
             num_W1, num_b1, time_W1, time_b1, proj_W, cn, ct, bvec, pe_pb,
             n_rows):
    rows_spec = pl.BlockSpec((NR, D), lambda i: (i, 0))
    full = lambda shape: pl.BlockSpec(shape, lambda i: (0,) * len(shape))
    return pl.pallas_call(
        _tc_body,
        grid=(n_rows // NR,),
        in_specs=[
            pl.BlockSpec((NR, 4), lambda i: (i, 0)),
            rows_spec, rows_spec, rows_spec,
            full((1, D // 2)), full((1, D // 2)),
            full((2, D // 2)), full((1, D // 2)),
            pl.BlockSpec((2 * D, D), lambda i: (0, 0)),
            full((D // 2, D)), full((D // 2, D)), full((1, D)),
            pl.BlockSpec((1, 1, D), lambda i: (i // (B // NR), 0, 0)),
        ],
        out_specs=pl.BlockSpec((NR, D), lambda i: (i, 0)),
        out_shape=jax.ShapeDtypeStruct((n_rows, D), jnp.float32),
    )(aux, tok_rows, act_rows, res_rows,
      num_W1, num_b1, time_W1, time_b1, proj_W, cn, ct, bvec, pe_pb)


def kernel(token_ids, activity_ids, resource_ids, numeric_features, time_features,
           token_table, activity_table, resource_table,
           num_W1, num_b1, num_W2, num_b2,
           time_W1, time_b1, time_W2, time_b2,
           proj_W, proj_b):
    tok_idx = token_ids.T.reshape(N).astype(jnp.int32)
    act_idx = activity_ids.T.reshape(N).astype(jnp.int32)
    res_idx = resource_ids.T.reshape(N).astype(jnp.int32)
    mask = (activity_ids.T > 0).astype(jnp.float32)[..., None]
    numT = numeric_features.transpose(1, 0, 2)
    timeT = time_features.transpose(1, 0, 2)
    aux = jnp.concatenate([mask, numT, timeT], axis=-1).reshape(N, 4)

    cn, ct, bvec = _fold(num_W2, time_W2, proj_W,
                         num_b2.reshape(1, D), time_b2.reshape(1, D))
    pe_pb = (jnp.asarray(_make_pe()) + proj_b[None, :]).reshape(S, 1, D)

    outs = []
    for k in range(K_SLICES):
        sl = slice(k * NSL, (k + 1) * NSL)
        tok_rows, act_rows, res_rows = _sc_gather3(
            token_table, activity_table, resource_table,
            tok_idx[sl], act_idx[sl], res_idx[sl], NSL)
        outs.append(_tc_fuse(
            aux[sl], tok_rows, act_rows, res_rows,
            num_W1.reshape(1, D // 2), num_b1.reshape(1, D // 2),
            time_W1, time_b1.reshape(1, D // 2),
            proj_W, cn, ct, bvec,
            pe_pb[k * S_SL:(k + 1) * S_SL], NSL))
    return jnp.concatenate(outs, axis=0).reshape(S, B, D)


# speedup vs baseline: 1.0771x; 1.0771x over previous
import functools

import numpy as np
import jax
import jax.numpy as jnp
from jax import lax
from jax.experimental import pallas as pl
from jax.experimental.pallas import tpu as pltpu
from jax.experimental.pallas import tpu_sc as plsc

B, S, V, D = 1024, 50, 100000, 128
N = B * S

_NC, _NS = 2, 16
NW = _NC * _NS
CH = 80

K_SLICES = 1
NSL = N // K_SLICES
S_SL = S // K_SLICES


def _make_pe():
    position = np.arange(S)[:, None].astype(np.float64)
    div_term = np.exp(np.arange(0, D, 2).astype(np.float64) * (-np.log(10000.0) / D))
    pe = np.zeros((S, D), dtype=np.float32)
    pe[:, 0::2] = np.sin(position * div_term)
    pe[:, 1::2] = np.cos(position * div_term)
    return pe


def _sc_gather3(tok_tab, act_tab, res_tab, tok_idx, act_idx, res_idx, n_rows):
    per_w = n_rows // NW
    nch = per_w // CH
    assert per_w % CH == 0 and nch % 2 == 0
    mesh = plsc.VectorSubcoreMesh(
        core_axis_name="c", subcore_axis_name="s",
        num_cores=_NC, num_subcores=_NS)

    @functools.partial(
        pl.kernel,
        out_type=(jax.ShapeDtypeStruct((n_rows, D), jnp.float32),) * 3,
        mesh=mesh,
        scratch_types=[
            pltpu.VMEM((per_w,), jnp.int32),
            pltpu.VMEM((CH, D), jnp.float32),
            pltpu.VMEM((CH, D), jnp.float32),
            pltpu.SemaphoreType.DMA,
            pltpu.SemaphoreType.DMA,
            pltpu.SemaphoreType.DMA,
        ],
    )
    def gather_k(tok_tab, act_tab, res_tab, tok_i, act_i, res_i,
                 o_tok, o_act, o_res, idx_v, buf_a, buf_b, sem_g, sem_sa, sem_sb):
        wid = lax.axis_index("s") * _NC + lax.axis_index("c")
        base = wid * per_w
        for tab, idx_hbm, out_hbm in ((tok_tab, tok_i, o_tok),
                                      (act_tab, act_i, o_act),
                                      (res_tab, res_i, o_res)):
            pltpu.sync_copy(idx_hbm.at[pl.ds(base, per_w)], idx_v)

            def gather(c, buf, tab=tab):
                pltpu.async_copy(
                    tab.at[idx_v.at[pl.ds(c * CH, CH)]], buf, sem_g).wait()

            def store_start(c, buf, sem, out_hbm=out_hbm):
                pltpu.async_copy(buf, out_hbm.at[pl.ds(base + c * CH, CH)], sem)

            def store_wait(c, buf, sem, out_hbm=out_hbm):
                pltpu.make_async_copy(
                    buf, out_hbm.at[pl.ds(base + c * CH, CH)], sem).wait()

            gather(0, buf_a)
            store_start(0, buf_a, sem_sa)
            gather(1, buf_b)
            store_start(1, buf_b, sem_sb)

            def pair(p, carry, tab=tab, out_hbm=out_hbm):
                c0 = 2 * p
                store_wait(c0 - 2, buf_a, sem_sa)
                gather(c0, buf_a)
                store_start(c0, buf_a, sem_sa)
                store_wait(c0 - 1, buf_b, sem_sb)
                gather(c0 + 1, buf_b)
                store_start(c0 + 1, buf_b, sem_sb)
                return carry

            lax.fori_loop(1, nch // 2, pair, 0)
            store_wait(nch - 2, buf_a, sem_sa)
            store_wait(nch - 1, buf_b, sem_sb)

    return gather_k(tok_tab, act_tab, res_tab, tok_idx, act_idx, res_idx)


def _fold_body(nW1, tW1, nb1, tb1, nW2, tW2, pW, nb2, tb2,
               w1_ref, b1_ref, c_ref, bvec_ref):
    wn = pW[2 * D:3 * D, :]
    wt = pW[3 * D:4 * D, :]
    z = jnp.zeros((1, D // 2), jnp.float32)
    w1_ref[...] = jnp.concatenate([
        jnp.concatenate([z, z], axis=1),
        jnp.concatenate([nW1[0:1, :], z], axis=1),
        jnp.concatenate([z, tW1[0:1, :]], axis=1),
        jnp.concatenate([z, tW1[1:2, :]], axis=1),
    ], axis=0)
    b1_ref[...] = jnp.concatenate([nb1[...], tb1[...]], axis=1)
    c_ref[...] = jnp.concatenate([
        jnp.dot(nW2[...], wn, preferred_element_type=jnp.float32),
        jnp.dot(tW2[...], wt, preferred_element_type=jnp.float32),
    ], axis=0)
    bvec_ref[...] = (jnp.dot(nb2[...], wn, preferred_element_type=jnp.float32)
                     + jnp.dot(tb2[...], wt, preferred_element_type=jnp.float32))


def _fold(num_W1, time_W1, num_b1, time_b1, num_W2, time_W2, proj_W,
          num_b2, time_b2):
    return pl.pallas_call(
        _fold_body,
        out_shape=(jax.ShapeDtypeStruct((4, D), jnp.float32),
                   jax.ShapeDtypeStruct((1, D), jnp.float32),
                   jax.ShapeDtypeStruct((D, D), jnp.float32),
                   jax.ShapeDtypeStruct((1, D), jnp.float32)),
    )(num_W1, time_W1, num_b1, time_b1, num_W2, time_W2, proj_W,
      num_b2, time_b2)


NR = 1024


def _tc_body(aux_ref, tok_ref, act_ref, res_ref, w1, b1, pW2, c_mat, bvec,
             pe_ref, out_ref):
    a = aux_ref[...]
    m = a[:, 0:1]
    h = jnp.maximum(
        jnp.dot(a, w1[...], preferred_element_type=jnp.float32) + b1[0:1, :],
        0.0)
    w = pW2[...]
    p = (jnp.dot(act_ref[...], w[0:D, :], preferred_element_type=jnp.float32)
         + jnp.dot(res_ref[...], w[D:2 * D, :], preferred_element_type=jnp.float32)
         + jnp.dot(h, c_mat[...], preferred_element_type=jnp.float32)
         + bvec[0:1, :])
    out_ref[...] = m * p + tok_ref[...] + pe_ref[0]


def _tc_fuse(aux, tok_rows, act_rows, res_rows,
             w1, b1, proj_W, c_mat, bvec, pe_pb, n_rows):
    rows_spec = pl.BlockSpec((NR, D), lambda i: (i, 0))
    full = lambda shape: pl.BlockSpec(shape, lambda i: (0,) * len(shape))
    return pl.pallas_call(
        _tc_body,
        grid=(n_rows // NR,),
        in_specs=[
            pl.BlockSpec((NR, 4), lambda i: (i, 0)),
            rows_spec, rows_spec, rows_spec,
            full((4, D)), full((1, D)),
            pl.BlockSpec((2 * D, D), lambda i: (0, 0)),
            full((D, D)), full((1, D)),
            pl.BlockSpec((1, 1, D), lambda i: (i // (B // NR), 0, 0)),
        ],
        out_specs=pl.BlockSpec((NR, D), lambda i: (i, 0)),
        out_shape=jax.ShapeDtypeStruct((n_rows, D), jnp.float32),
    )(aux, tok_rows, act_rows, res_rows, w1, b1, proj_W, c_mat, bvec, pe_pb)


def kernel(token_ids, activity_ids, resource_ids, numeric_features, time_features,
           token_table, activity_table, resource_table,
           num_W1, num_b1, num_W2, num_b2,
           time_W1, time_b1, time_W2, time_b2,
           proj_W, proj_b):
    tok_idx = token_ids.T.reshape(N).astype(jnp.int32)
    act_idx = activity_ids.T.reshape(N).astype(jnp.int32)
    res_idx = resource_ids.T.reshape(N).astype(jnp.int32)
    mask = (activity_ids.T > 0).astype(jnp.float32)[..., None]
    numT = numeric_features.transpose(1, 0, 2)
    timeT = time_features.transpose(1, 0, 2)
    aux = jnp.concatenate([mask, numT, timeT], axis=-1).reshape(N, 4)

    w1, b1, c_mat, bvec = _fold(
        num_W1.reshape(1, D // 2), time_W1,
        num_b1.reshape(1, D // 2), time_b1.reshape(1, D // 2),
        num_W2, time_W2, proj_W,
        num_b2.reshape(1, D), time_b2.reshape(1, D))
    pe_pb = (jnp.asarray(_make_pe()) + proj_b[None, :]).reshape(S, 1, D)

    outs = []
    for k in range(K_SLICES):
        sl = slice(k * NSL, (k + 1) * NSL)
        tok_rows, act_rows, res_rows = _sc_gather3(
            token_table, activity_table, resource_table,
            tok_idx[sl], act_idx[sl], res_idx[sl], NSL)
        outs.append(_tc_fuse(
            aux[sl], tok_rows, act_rows, res_rows,
            w1, b1, proj_W, c_mat, bvec,
            pe_pb[k * S_SL:(k + 1) * S_SL], NSL))
    return jnp.concatenate(outs, axis=0).reshape(S, B, D)


# speedup vs baseline: 1.1721x; 1.0882x over previous
import functools

import numpy as np
import jax
import jax.numpy as jnp
from jax import lax
from jax.experimental import pallas as pl
from jax.experimental.pallas import tpu as pltpu
from jax.experimental.pallas import tpu_sc as plsc

B, S, V, D = 1024, 50, 100000, 128
N = B * S

_NC, _NS = 2, 16
NW = _NC * _NS
CH = 80

K_SLICES = 1
NSL = N // K_SLICES
S_SL = S // K_SLICES


def _make_pe():
    position = np.arange(S)[:, None].astype(np.float64)
    div_term = np.exp(np.arange(0, D, 2).astype(np.float64) * (-np.log(10000.0) / D))
    pe = np.zeros((S, D), dtype=np.float32)
    pe[:, 0::2] = np.sin(position * div_term)
    pe[:, 1::2] = np.cos(position * div_term)
    return pe


def _sc_gather3(tok_tab, act_tab, res_tab, tok_idx, act_idx, res_idx, n_rows):
    per_w = n_rows // NW
    nch = per_w // CH
    assert per_w % CH == 0 and nch % 2 == 0
    mesh = plsc.VectorSubcoreMesh(
        core_axis_name="c", subcore_axis_name="s",
        num_cores=_NC, num_subcores=_NS)

    @functools.partial(
        pl.kernel,
        out_type=(jax.ShapeDtypeStruct((n_rows, D), jnp.float32),) * 3,
        mesh=mesh,
        scratch_types=[
            pltpu.VMEM((per_w,), jnp.int32),
            pltpu.VMEM((CH, D), jnp.float32),
            pltpu.VMEM((CH, D), jnp.float32),
            pltpu.SemaphoreType.DMA,
            pltpu.SemaphoreType.DMA,
            pltpu.SemaphoreType.DMA,
        ],
    )
    def gather_k(tok_tab, act_tab, res_tab, tok_i, act_i, res_i,
                 o_tok, o_act, o_res, idx_v, buf_a, buf_b, sem_g, sem_sa, sem_sb):
        wid = lax.axis_index("s") * _NC + lax.axis_index("c")
        base = wid * per_w
        for tab, idx_hbm, out_hbm in ((tok_tab, tok_i, o_tok),
                                      (act_tab, act_i, o_act),
                                      (res_tab, res_i, o_res)):
            pltpu.sync_copy(idx_hbm.at[pl.ds(base, per_w)], idx_v)

            def gather(c, buf, tab=tab):
                pltpu.async_copy(
                    tab.at[idx_v.at[pl.ds(c * CH, CH)]], buf, sem_g).wait()

            def store_start(c, buf, sem, out_hbm=out_hbm):
                pltpu.async_copy(buf, out_hbm.at[pl.ds(base + c * CH, CH)], sem)

            def store_wait(c, buf, sem, out_hbm=out_hbm):
                pltpu.make_async_copy(
                    buf, out_hbm.at[pl.ds(base + c * CH, CH)], sem).wait()

            gather(0, buf_a)
            store_start(0, buf_a, sem_sa)
            gather(1, buf_b)
            store_start(1, buf_b, sem_sb)

            def pair(p, carry, tab=tab, out_hbm=out_hbm):
                c0 = 2 * p
                store_wait(c0 - 2, buf_a, sem_sa)
                gather(c0, buf_a)
                store_start(c0, buf_a, sem_sa)
                store_wait(c0 - 1, buf_b, sem_sb)
                gather(c0 + 1, buf_b)
                store_start(c0 + 1, buf_b, sem_sb)
                return carry

            lax.fori_loop(1, nch // 2, pair, 0)
            store_wait(nch - 2, buf_a, sem_sa)
            store_wait(nch - 1, buf_b, sem_sb)

    return gather_k(tok_tab, act_tab, res_tab, tok_idx, act_idx, res_idx)


def _fold_body(nW1, tW1, nb1, tb1, nW2, tW2, pW, nb2, tb2,
               w1_ref, b1_ref, c_ref, bvec_ref):
    wn = pW[2 * D:3 * D, :]
    wt = pW[3 * D:4 * D, :]
    z = jnp.zeros((1, D // 2), jnp.float32)
    w1_ref[...] = jnp.concatenate([
        jnp.concatenate([z, z], axis=1),
        jnp.concatenate([nW1[0:1, :], z], axis=1),
        jnp.concatenate([z, tW1[0:1, :]], axis=1),
        jnp.concatenate([z, tW1[1:2, :]], axis=1),
    ], axis=0)
    b1_ref[...] = jnp.concatenate([nb1[...], tb1[...]], axis=1)
    c_ref[...] = jnp.concatenate([
        jnp.dot(nW2[...], wn, preferred_element_type=jnp.float32),
        jnp.dot(tW2[...], wt, preferred_element_type=jnp.float32),
    ], axis=0)
    bvec_ref[...] = (jnp.dot(nb2[...], wn, preferred_element_type=jnp.float32)
                     + jnp.dot(tb2[...], wt, preferred_element_type=jnp.float32))


def _fold(num_W1, time_W1, num_b1, time_b1, num_W2, time_W2, proj_W,
          num_b2, time_b2):
    return pl.pallas_call(
        _fold_body,
        out_shape=(jax.ShapeDtypeStruct((4, D), jnp.float32),
                   jax.ShapeDtypeStruct((1, D), jnp.float32),
                   jax.ShapeDtypeStruct((D, D), jnp.float32),
                   jax.ShapeDtypeStruct((1, D), jnp.float32)),
    )(num_W1, time_W1, num_b1, time_b1, num_W2, time_W2, proj_W,
      num_b2, time_b2)


NR = 2048
RPB = NR // B


def _tc_body(aux_ref, tok_ref, act_ref, res_ref, w1, b1, pW2, c_mat, bvec,
             pe_ref, out_ref):
    a = aux_ref[...]
    m = a[:, 0:1]
    h = jnp.maximum(
        jnp.dot(a, w1[...], preferred_element_type=jnp.float32) + b1[0:1, :],
        0.0)
    w = pW2[...]
    p = (jnp.dot(act_ref[...], w[0:D, :], preferred_element_type=jnp.float32)
         + jnp.dot(res_ref[...], w[D:2 * D, :], preferred_element_type=jnp.float32)
         + jnp.dot(h, c_mat[...], preferred_element_type=jnp.float32)
         + bvec[0:1, :])
    pe = jnp.broadcast_to(pe_ref[0][:, None, :], (RPB, B, D)).reshape(NR, D)
    out_ref[...] = m * p + tok_ref[...] + pe


def _tc_fuse(aux, tok_rows, act_rows, res_rows,
             w1, b1, proj_W, c_mat, bvec, pe_pb, n_rows):
    rows_spec = pl.BlockSpec((NR, D), lambda i: (i, 0))
    full = lambda shape: pl.BlockSpec(shape, lambda i: (0,) * len(shape))
    return pl.pallas_call(
        _tc_body,
        grid=(n_rows // NR,),
        in_specs=[
            pl.BlockSpec((NR, 4), lambda i: (i, 0)),
            rows_spec, rows_spec, rows_spec,
            full((4, D)), full((1, D)),
            pl.BlockSpec((2 * D, D), lambda i: (0, 0)),
            full((D, D)), full((1, D)),
            pl.BlockSpec((1, RPB, D), lambda i: (i, 0, 0)),
        ],
        out_specs=pl.BlockSpec((NR, D), lambda i: (i, 0)),
        out_shape=jax.ShapeDtypeStruct((n_rows, D), jnp.float32),
    )(aux, tok_rows, act_rows, res_rows, w1, b1, proj_W, c_mat, bvec, pe_pb)


def kernel(token_ids, activity_ids, resource_ids, numeric_features, time_features,
           token_table, activity_table, resource_table,
           num_W1, num_b1, num_W2, num_b2,
           time_W1, time_b1, time_W2, time_b2,
           proj_W, proj_b):
    tok_idx = token_ids.T.reshape(N).astype(jnp.int32)
    act_idx = activity_ids.T.reshape(N).astype(jnp.int32)
    res_idx = resource_ids.T.reshape(N).astype(jnp.int32)
    mask = (activity_ids.T > 0).astype(jnp.float32)[..., None]
    numT = numeric_features.transpose(1, 0, 2)
    timeT = time_features.transpose(1, 0, 2)
    aux = jnp.concatenate([mask, numT, timeT], axis=-1).reshape(N, 4)

    w1, b1, c_mat, bvec = _fold(
        num_W1.reshape(1, D // 2), time_W1,
        num_b1.reshape(1, D // 2), time_b1.reshape(1, D // 2),
        num_W2, time_W2, proj_W,
        num_b2.reshape(1, D), time_b2.reshape(1, D))
    pe_pb = (jnp.asarray(_make_pe()) + proj_b[None, :]).reshape(S // RPB, RPB, D)

    outs = []
    for k in range(K_SLICES):
        sl = slice(k * NSL, (k + 1) * NSL)
        tok_rows, act_rows, res_rows = _sc_gather3(
            token_table, activity_table, resource_table,
            tok_idx[sl], act_idx[sl], res_idx[sl], NSL)
        outs.append(_tc_fuse(
            aux[sl], tok_rows, act_rows, res_rows,
            w1, b1, proj_W, c_mat, bvec,
            pe_pb[k * (S_SL // RPB):(k + 1) * (S_SL // RPB)], NSL))
    return jnp.concatenate(outs, axis=0).reshape(S, B, D)


# speedup vs baseline: 1.2177x; 1.0389x over previous
import functools

import numpy as np
import jax
import jax.numpy as jnp
from jax import lax
from jax.experimental import pallas as pl
from jax.experimental.pallas import tpu as pltpu
from jax.experimental.pallas import tpu_sc as plsc

B, S, V, D = 1024, 50, 100000, 128
N = B * S

_NC, _NS = 2, 16
NW = _NC * _NS
CH = 80

K_SLICES = 1
NSL = N // K_SLICES
S_SL = S // K_SLICES


def _make_pe():
    position = np.arange(S)[:, None].astype(np.float64)
    div_term = np.exp(np.arange(0, D, 2).astype(np.float64) * (-np.log(10000.0) / D))
    pe = np.zeros((S, D), dtype=np.float32)
    pe[:, 0::2] = np.sin(position * div_term)
    pe[:, 1::2] = np.cos(position * div_term)
    return pe


def _sc_gather3(tok_tab, act_tab, res_tab, tok_idx, act_idx, res_idx, n_rows):
    per_w = n_rows // NW
    nch = per_w // CH
    assert per_w % CH == 0 and nch % 2 == 0
    mesh = plsc.VectorSubcoreMesh(
        core_axis_name="c", subcore_axis_name="s",
        num_cores=_NC, num_subcores=_NS)

    @functools.partial(
        pl.kernel,
        out_type=(jax.ShapeDtypeStruct((n_rows, D), jnp.float32),) * 3,
        mesh=mesh,
        scratch_types=[
            pltpu.VMEM((per_w,), jnp.int32),
            pltpu.VMEM((CH, D), jnp.float32),
            pltpu.VMEM((CH, D), jnp.float32),
            pltpu.SemaphoreType.DMA,
            pltpu.SemaphoreType.DMA,
            pltpu.SemaphoreType.DMA,
        ],
    )
    def gather_k(tok_tab, act_tab, res_tab, tok_i, act_i, res_i,
                 o_tok, o_act, o_res, idx_v, buf_a, buf_b, sem_g, sem_sa, sem_sb):
        wid = lax.axis_index("s") * _NC + lax.axis_index("c")
        base = wid * per_w
        for tab, idx_hbm, out_hbm in ((tok_tab, tok_i, o_tok),
                                      (act_tab, act_i, o_act),
                                      (res_tab, res_i, o_res)):
            pltpu.sync_copy(idx_hbm.at[pl.ds(base, per_w)], idx_v)

            def gather(c, buf, tab=tab):
                pltpu.async_copy(
                    tab.at[idx_v.at[pl.ds(c * CH, CH)]], buf, sem_g).wait()

            def store_start(c, buf, sem, out_hbm=out_hbm):
                pltpu.async_copy(buf, out_hbm.at[pl.ds(base + c * CH, CH)], sem)

            def store_wait(c, buf, sem, out_hbm=out_hbm):
                pltpu.make_async_copy(
                    buf, out_hbm.at[pl.ds(base + c * CH, CH)], sem).wait()

            gather(0, buf_a)
            store_start(0, buf_a, sem_sa)
            gather(1, buf_b)
            store_start(1, buf_b, sem_sb)

            def pair(p, carry, tab=tab, out_hbm=out_hbm):
                c0 = 2 * p
                store_wait(c0 - 2, buf_a, sem_sa)
                gather(c0, buf_a)
                store_start(c0, buf_a, sem_sa)
                store_wait(c0 - 1, buf_b, sem_sb)
                gather(c0 + 1, buf_b)
                store_start(c0 + 1, buf_b, sem_sb)
                return carry

            lax.fori_loop(1, nch // 2, pair, 0)
            store_wait(nch - 2, buf_a, sem_sa)
            store_wait(nch - 1, buf_b, sem_sb)

    return gather_k(tok_tab, act_tab, res_tab, tok_idx, act_idx, res_idx)


def _fold_body(nW1, tW1, nb1, tb1, nW2, tW2, pW, nb2, tb2,
               w1_ref, b1_ref, c_ref, bvec_ref):
    wn = pW[2 * D:3 * D, :]
    wt = pW[3 * D:4 * D, :]
    z = jnp.zeros((1, D // 2), jnp.float32)
    w1_ref[...] = jnp.concatenate([
        jnp.concatenate([z, z], axis=1),
        jnp.concatenate([nW1[0:1, :], z], axis=1),
        jnp.concatenate([z, tW1[0:1, :]], axis=1),
        jnp.concatenate([z, tW1[1:2, :]], axis=1),
    ], axis=0)
    b1_ref[...] = jnp.concatenate([nb1[...], tb1[...]], axis=1)
    c_ref[...] = jnp.concatenate([
        jnp.dot(nW2[...], wn, preferred_element_type=jnp.float32),
        jnp.dot(tW2[...], wt, preferred_element_type=jnp.float32),
    ], axis=0)
    bvec_ref[...] = (jnp.dot(nb2[...], wn, preferred_element_type=jnp.float32)
                     + jnp.dot(tb2[...], wt, preferred_element_type=jnp.float32))


def _fold(num_W1, time_W1, num_b1, time_b1, num_W2, time_W2, proj_W,
          num_b2, time_b2):
    return pl.pallas_call(
        _fold_body,
        out_shape=(jax.ShapeDtypeStruct((4, D), jnp.float32),
                   jax.ShapeDtypeStruct((1, D), jnp.float32),
                   jax.ShapeDtypeStruct((D, D), jnp.float32),
                   jax.ShapeDtypeStruct((1, D), jnp.float32)),
    )(num_W1, time_W1, num_b1, time_b1, num_W2, time_W2, proj_W,
      num_b2, time_b2)


NR = 5120
RPB = NR // B


def _tc_body(aux_ref, tok_ref, act_ref, res_ref, w1, b1, pW2, c_mat, bvec,
             pe_ref, out_ref):
    a = aux_ref[...]
    m = a[:, 0:1]
    h = jnp.maximum(
        jnp.dot(a, w1[...], preferred_element_type=jnp.float32) + b1[0:1, :],
        0.0)
    w = pW2[...]
    p = (jnp.dot(act_ref[...], w[0:D, :], preferred_element_type=jnp.float32)
         + jnp.dot(res_ref[...], w[D:2 * D, :], preferred_element_type=jnp.float32)
         + jnp.dot(h, c_mat[...], preferred_element_type=jnp.float32)
         + bvec[0:1, :])
    pe = jnp.broadcast_to(pe_ref[0][:, None, :], (RPB, B, D)).reshape(NR, D)
    out_ref[...] = m * p + tok_ref[...] + pe


def _tc_fuse(aux, tok_rows, act_rows, res_rows,
             w1, b1, proj_W, c_mat, bvec, pe_pb, n_rows):
    rows_spec = pl.BlockSpec((NR, D), lambda i: (i, 0))
    full = lambda shape: pl.BlockSpec(shape, lambda i: (0,) * len(shape))
    return pl.pallas_call(
        _tc_body,
        grid=(n_rows // NR,),
        in_specs=[
            pl.BlockSpec((NR, 4), lambda i: (i, 0)),
            rows_spec, rows_spec, rows_spec,
            full((4, D)), full((1, D)),
            pl.BlockSpec((2 * D, D), lambda i: (0, 0)),
            full((D, D)), full((1, D)),
            pl.BlockSpec((1, RPB, D), lambda i: (i, 0, 0)),
        ],
        out_specs=pl.BlockSpec((NR, D), lambda i: (i, 0)),
        out_shape=jax.ShapeDtypeStruct((n_rows, D), jnp.float32),
    )(aux, tok_rows, act_rows, res_rows, w1, b1, proj_W, c_mat, bvec, pe_pb)


def kernel(token_ids, activity_ids, resource_ids, numeric_features, time_features,
           token_table, activity_table, resource_table,
           num_W1, num_b1, num_W2, num_b2,
           time_W1, time_b1, time_W2, time_b2,
           proj_W, proj_b):
    tok_idx = token_ids.T.reshape(N).astype(jnp.int32)
    act_idx = activity_ids.T.reshape(N).astype(jnp.int32)
    res_idx = resource_ids.T.reshape(N).astype(jnp.int32)
    mask = (activity_ids.T > 0).astype(jnp.float32)[..., None]
    numT = numeric_features.transpose(1, 0, 2)
    timeT = time_features.transpose(1, 0, 2)
    aux = jnp.concatenate([mask, numT, timeT], axis=-1).reshape(N, 4)

    w1, b1, c_mat, bvec = _fold(
        num_W1.reshape(1, D // 2), time_W1,
        num_b1.reshape(1, D // 2), time_b1.reshape(1, D // 2),
        num_W2, time_W2, proj_W,
        num_b2.reshape(1, D), time_b2.reshape(1, D))
    pe_pb = (jnp.asarray(_make_pe()) + proj_b[None, :]).reshape(S // RPB, RPB, D)

    outs = []
    for k in range(K_SLICES):
        sl = slice(k * NSL, (k + 1) * NSL)
        tok_rows, act_rows, res_rows = _sc_gather3(
            token_table, activity_table, resource_table,
            tok_idx[sl], act_idx[sl], res_idx[sl], NSL)
        outs.append(_tc_fuse(
            aux[sl], tok_rows, act_rows, res_rows,
            w1, b1, proj_W, c_mat, bvec,
            pe_pb[k * (S_SL // RPB):(k + 1) * (S_SL // RPB)], NSL))
    return jnp.concatenate(outs, axis=0).reshape(S, B, D)
